# baseline (speedup 1.0000x reference)
import functools

import jax
import jax.numpy as jnp
from jax import lax
from jax.experimental import pallas as pl
from jax.experimental.pallas import tpu as pltpu
from jax.experimental.pallas import tpu_sc as plsc

_LN2 = 0.6931471805599453


def _ssp(x):
    return jnp.maximum(x, 0.0) + jnp.log1p(jnp.exp(-jnp.abs(x))) - _LN2


def _filters_body(d_ref, c_ref, g_ref, w1_ref, b1_ref, w2_ref, b2_ref, o_ref):
    d = d_ref[...].reshape(-1, 1)
    ex = jnp.exp(-g_ref[...] * (d - c_ref[...]) ** 2)
    h = jnp.dot(ex, w1_ref[...], preferred_element_type=jnp.float32) + b1_ref[...]
    h = _ssp(h)
    f = jnp.dot(h, w2_ref[...], preferred_element_type=jnp.float32) + b2_ref[...]
    f = _ssp(f)
    Dh = f.shape[1] // 2
    lo = jax.lax.bitcast_convert_type(
        f[:, :Dh].astype(jnp.bfloat16), jnp.uint16).astype(jnp.uint32)
    hi = jax.lax.bitcast_convert_type(
        f[:, Dh:].astype(jnp.bfloat16), jnp.uint16).astype(jnp.uint32)
    o_ref[...] = jax.lax.bitcast_convert_type(lo | (hi << 16), jnp.int32)


def _compute_filters(dist, centers, gamma, W1, b1, W2, b2, perm):
    E = dist.shape[0]
    R = centers.shape[0]
    D = W1.shape[1]
    BE = next(b for b in range(3328, 127, -128) if E % b == 0)
    return pl.pallas_call(
        _filters_body,
        grid=(E // BE,),
        in_specs=[
            pl.BlockSpec((1, BE), lambda i: (0, i)),
            pl.BlockSpec((1, R), lambda i: (0, 0)),
            pl.BlockSpec((1, R), lambda i: (0, 0)),
            pl.BlockSpec((R, D), lambda i: (0, 0)),
            pl.BlockSpec((1, D), lambda i: (0, 0)),
            pl.BlockSpec((D, D), lambda i: (0, 0)),
            pl.BlockSpec((1, D), lambda i: (0, 0)),
        ],
        out_specs=pl.BlockSpec((BE, D // 2), lambda i: (i, 0)),
        out_shape=jax.ShapeDtypeStruct((E, D // 2), jnp.int32),
    )(
        dist.reshape(1, E),
        centers.reshape(1, R),
        gamma.reshape(1, R),
        W1,
        b1.reshape(1, D),
        W2[:, perm],
        b2[perm].reshape(1, D),
    )


def _sc_gather_mult_segsum(af, filters, idx, seg):
    N, D = af.shape
    E = filters.shape[0]
    NC, NS, L = 2, 16, 16
    NW = NC * NS
    EW = E // NW
    C = next(c for c in range(128, 7, -8) if EW % c == 0)
    assert E % (NW * C) == 0 and D % L == 0
    NF = EW // C
    RZ = (N // NS) // 8 * 8
    NREST = N - RZ * NS
    assert EW % 8 == 0
    mesh = plsc.VectorSubcoreMesh(
        core_axis_name="c", subcore_axis_name="s", num_cores=NC, num_subcores=NS
    )

    @functools.partial(
        pl.kernel,
        out_type=jax.ShapeDtypeStruct((NC, N, D), jnp.float32),
        mesh=mesh,
        scratch_types=[
            pltpu.VMEM((C,), jnp.int32),
            pltpu.VMEM((C,), jnp.int32),
            pltpu.VMEM((C,), jnp.int32),
            pltpu.VMEM((C,), jnp.int32),
            pltpu.VMEM((C, D), jnp.float32),
            pltpu.VMEM((C, D), jnp.float32),
            pltpu.VMEM((C, D // 2), jnp.int32),
            pltpu.VMEM((C, D // 2), jnp.int32),
            pltpu.VMEM_SHARED((N, D), jnp.float32),
            pltpu.SemaphoreType.DMA,
            pltpu.SemaphoreType.DMA,
            pltpu.SemaphoreType.DMA,
            pltpu.SemaphoreType.DMA,
            pltpu.SemaphoreType.DMA,
            pltpu.SemaphoreType.DMA,
            pltpu.SemaphoreType.DMA,
            pltpu.SemaphoreType.DMA,
        ],
    )
    def k(af_hbm, filt_hbm, idx_hbm, seg_hbm, out_hbm,
          ibuf0, ibuf1, sbuf0, sbuf1, rows0, rows1, filt0, filt1, acc,
          gsem0, gsem1, fsem0, fsem1, isem0, isem1, ssem0, ssem1):
        ibuf = (ibuf0, ibuf1)
        sbuf = (sbuf0, sbuf1)
        rows = (rows0, rows1)
        filt = (filt0, filt1)
        gsem = (gsem0, gsem1)
        fsem = (fsem0, fsem1)
        isem = (isem0, isem1)
        ssem = (ssem0, ssem1)
        cid = lax.axis_index("c")
        sid = lax.axis_index("s")
        wid = cid * NS + sid
        ebase = wid * EW

        @pl.loop(0, C)
        def _z(r):
            for c8 in range(D // L):
                rows0[r, pl.ds(c8 * L, L)] = jnp.zeros((L,), jnp.float32)

        nz = RZ // C
        rz = RZ - nz * C

        @pl.loop(0, nz)
        def _zc(kk):
            pltpu.sync_copy(rows0, acc.at[pl.ds(sid * RZ + kk * C, C)])

        if rz > 0:
            pltpu.sync_copy(
                rows0.at[pl.ds(0, rz)], acc.at[pl.ds(sid * RZ + nz * C, rz)]
            )
        if NREST > 0:
            @pl.when(sid == 0)
            def _zrest():
                pltpu.sync_copy(
                    rows0.at[pl.ds(0, NREST)], acc.at[pl.ds(RZ * NS, NREST)]
                )
        plsc.subcore_barrier()

        def issue_idx(g, b):
            pltpu.async_copy(idx_hbm.at[pl.ds(ebase + g * C, C)], ibuf[b], isem[b])

        def issue_seg(g, b):
            pltpu.async_copy(seg_hbm.at[pl.ds(ebase + g * C, C)], sbuf[b], ssem[b])

        def issue_data(g, b):
            pltpu.async_copy(af_hbm.at[ibuf[b]], rows[b], gsem[b])
            pltpu.async_copy(
                filt_hbm.at[pl.ds(ebase + g * C, C)], filt[b], fsem[b]
            )

        def wait_idx(b):
            pltpu.make_async_copy(idx_hbm.at[pl.ds(0, C)], ibuf[b], isem[b]).wait()

        def wait_seg(b):
            pltpu.make_async_copy(seg_hbm.at[pl.ds(0, C)], sbuf[b], ssem[b]).wait()

        def wait_data(b):
            pltpu.make_async_copy(af_hbm.at[pl.ds(0, C)], rows[b], gsem[b]).wait()
            pltpu.make_async_copy(filt_hbm.at[pl.ds(0, C)], filt[b], fsem[b]).wait()

        def multiply(b):
            @pl.loop(0, C)
            def _m(r):
                for q in range(D // (2 * L)):
                    w = filt[b][r, pl.ds(q * L, L)]
                    fpair = (
                        jax.lax.bitcast_convert_type(w << 16, jnp.float32),
                        jax.lax.bitcast_convert_type(
                            w & jnp.int32(-65536), jnp.float32),
                    )
                    for s in range(2):
                        sl = pl.ds(q * 2 * L + s * L, L)
                        rows[b][r, sl] = rows[b][r, sl] * fpair[s]

        for b in range(2):
            issue_idx(b, b)
            issue_seg(b, b)
        for b in range(2):
            wait_idx(b)
            issue_data(b, b)

        NF2 = NF - (NF % 2)

        @pl.loop(0, NF2, step=2)
        def _main(g):
            for b in range(2):
                gg = g + b
                nxt = jnp.minimum(gg + 2, NF - 1)
                wait_data(b)
                issue_idx(nxt, b)
                multiply(b)
                wait_seg(b)
                pltpu.sync_copy(rows[b], acc.at[sbuf[b]], add=True)
                issue_seg(nxt, b)
                wait_idx(b)
                issue_data(nxt, b)

        if NF % 2:
            wait_data(0)
            multiply(0)
            wait_seg(0)
            pltpu.sync_copy(rows[0], acc.at[sbuf[0]], add=True)
            wait_data(1)
            wait_seg(1)
        else:
            for b in range(2):
                wait_data(b)
                wait_seg(b)

        plsc.subcore_barrier()

        pltpu.sync_copy(
            acc.at[pl.ds(sid * RZ, RZ)], out_hbm.at[cid, pl.ds(sid * RZ, RZ)]
        )
        if NREST > 0:
            @pl.when(sid == 0)
            def _rb():
                pltpu.sync_copy(
                    acc.at[pl.ds(RZ * NS, NREST)],
                    out_hbm.at[cid, pl.ds(RZ * NS, NREST)],
                )

    return k(af, filters, idx, seg)


def _add_body(p_ref, q_ref, o_ref):
    o_ref[...] = (p_ref[0] + p_ref[1]) + (q_ref[0] + q_ref[1])


def _add_partials(p, q):
    _, N, D = p.shape
    BN = 2000
    assert N % BN == 0
    spec = pl.BlockSpec((2, BN, D), lambda i: (0, i, 0))
    return pl.pallas_call(
        _add_body,
        grid=(N // BN,),
        in_specs=[spec, spec],
        out_specs=pl.BlockSpec((BN, D), lambda i: (i, 0)),
        out_shape=jax.ShapeDtypeStruct((N, D), jnp.float32),
    )(p, q)


def kernel(atom_features, distances, rbf_centers, rbf_gamma, W1, b1, W2, b2, idx_j, seg_i):
    B, N, D = atom_features.shape
    E = distances.shape[1]
    af = atom_features.reshape(N, D)
    dist = distances.reshape(E)
    idx = idx_j.astype(jnp.int32)
    seg = seg_i.astype(jnp.int32)
    import numpy as _np
    ch_lo = _np.array([32 * (m // 16) + m % 16 for m in range(D // 2)])
    perm = _np.concatenate([ch_lo, ch_lo + 16])

    H = E // 2
    parts = []
    for k in range(2):
        sl = slice(k * H, (k + 1) * H)
        f_k = _compute_filters(dist[sl], rbf_centers, rbf_gamma, W1, b1, W2, b2, perm)
        parts.append(_sc_gather_mult_segsum(af, f_k, idx[sl], seg[sl]))
    out = _add_partials(parts[0], parts[1])
    return out.reshape(B, N, D)


# speedup vs baseline: 1.0857x; 1.0857x over previous
import functools

import jax
import jax.numpy as jnp
from jax import lax
from jax.experimental import pallas as pl
from jax.experimental.pallas import tpu as pltpu
from jax.experimental.pallas import tpu_sc as plsc

_LN2 = 0.6931471805599453


def _ssp(x):
    return jnp.maximum(x, 0.0) + jnp.log1p(jnp.exp(-jnp.abs(x))) - _LN2


def _filters_body(d_ref, c_ref, g_ref, w1_ref, b1_ref, w2_ref, b2_ref, o_ref):
    d = d_ref[...].reshape(-1, 1)
    ex = jnp.exp(-g_ref[...] * (d - c_ref[...]) ** 2)
    h = jnp.dot(ex, w1_ref[...], preferred_element_type=jnp.float32) + b1_ref[...]
    h = _ssp(h)
    f = jnp.dot(h, w2_ref[...], preferred_element_type=jnp.float32) + b2_ref[...]
    o_ref[...] = _ssp(f)


def _compute_filters(dist, centers, gamma, W1, b1, W2, b2):
    E = dist.shape[1]
    R = centers.shape[0]
    D = W1.shape[1]
    BE = next(b for b in range(3328, 127, -128) if E % b == 0)
    return pl.pallas_call(
        _filters_body,
        grid=(E // BE,),
        in_specs=[
            pl.BlockSpec((1, BE), lambda i: (0, i)),
            pl.BlockSpec((1, R), lambda i: (0, 0)),
            pl.BlockSpec((1, R), lambda i: (0, 0)),
            pl.BlockSpec((R, D), lambda i: (0, 0)),
            pl.BlockSpec((1, D), lambda i: (0, 0)),
            pl.BlockSpec((D, D), lambda i: (0, 0)),
            pl.BlockSpec((1, D), lambda i: (0, 0)),
        ],
        out_specs=pl.BlockSpec((BE, D), lambda i: (i, 0)),
        out_shape=jax.ShapeDtypeStruct((E, D), jnp.float32),
    )(
        dist,
        centers.reshape(1, R),
        gamma.reshape(1, R),
        W1,
        b1.reshape(1, D),
        W2,
        b2.reshape(1, D),
    )


def _sc_gather_mult_segsum(af, filters, idx, seg):
    N, D = af.shape
    E = filters.shape[0]
    NC, NS, L = 2, 16, 16
    NW = NC * NS
    EW = E // NW
    C = next(c for c in range(128, 7, -8) if EW % c == 0)
    assert E % (NW * C) == 0 and D % L == 0
    NF = EW // C
    RZ = (N // NS) // 8 * 8
    NREST = N - RZ * NS
    assert EW % 8 == 0
    mesh = plsc.VectorSubcoreMesh(
        core_axis_name="c", subcore_axis_name="s", num_cores=NC, num_subcores=NS
    )

    @functools.partial(
        pl.kernel,
        out_type=jax.ShapeDtypeStruct((NC, N, D), jnp.float32),
        mesh=mesh,
        scratch_types=[
            pltpu.VMEM((C,), jnp.int32),
            pltpu.VMEM((C,), jnp.int32),
            pltpu.VMEM((C,), jnp.int32),
            pltpu.VMEM((C,), jnp.int32),
            pltpu.VMEM((C, D), jnp.float32),
            pltpu.VMEM((C, D), jnp.float32),
            pltpu.VMEM((C, D), jnp.float32),
            pltpu.VMEM((C, D), jnp.float32),
            pltpu.VMEM_SHARED((N, D), jnp.float32),
            pltpu.SemaphoreType.DMA,
            pltpu.SemaphoreType.DMA,
            pltpu.SemaphoreType.DMA,
            pltpu.SemaphoreType.DMA,
            pltpu.SemaphoreType.DMA,
            pltpu.SemaphoreType.DMA,
            pltpu.SemaphoreType.DMA,
            pltpu.SemaphoreType.DMA,
        ],
    )
    def k(af_hbm, filt_hbm, idx_hbm, seg_hbm, out_hbm,
          ibuf0, ibuf1, sbuf0, sbuf1, rows0, rows1, filt0, filt1, acc,
          gsem0, gsem1, fsem0, fsem1, isem0, isem1, ssem0, ssem1):
        ibuf = (ibuf0, ibuf1)
        sbuf = (sbuf0, sbuf1)
        rows = (rows0, rows1)
        filt = (filt0, filt1)
        gsem = (gsem0, gsem1)
        fsem = (fsem0, fsem1)
        isem = (isem0, isem1)
        ssem = (ssem0, ssem1)
        cid = lax.axis_index("c")
        sid = lax.axis_index("s")
        wid = cid * NS + sid
        ebase = wid * EW

        @pl.loop(0, C)
        def _z(r):
            for c8 in range(D // L):
                rows0[r, pl.ds(c8 * L, L)] = jnp.zeros((L,), jnp.float32)

        nz = RZ // C
        rz = RZ - nz * C

        @pl.loop(0, nz)
        def _zc(kk):
            pltpu.sync_copy(rows0, acc.at[pl.ds(sid * RZ + kk * C, C)])

        if rz > 0:
            pltpu.sync_copy(
                rows0.at[pl.ds(0, rz)], acc.at[pl.ds(sid * RZ + nz * C, rz)]
            )
        if NREST > 0:
            @pl.when(sid == 0)
            def _zrest():
                pltpu.sync_copy(
                    rows0.at[pl.ds(0, NREST)], acc.at[pl.ds(RZ * NS, NREST)]
                )
        plsc.subcore_barrier()

        def issue_idx(g, b):
            pltpu.async_copy(idx_hbm.at[pl.ds(ebase + g * C, C)], ibuf[b], isem[b])

        def issue_seg(g, b):
            pltpu.async_copy(seg_hbm.at[pl.ds(ebase + g * C, C)], sbuf[b], ssem[b])

        def issue_data(g, b):
            pltpu.async_copy(af_hbm.at[ibuf[b]], rows[b], gsem[b])
            pltpu.async_copy(
                filt_hbm.at[pl.ds(ebase + g * C, C)], filt[b], fsem[b]
            )

        def wait_idx(b):
            pltpu.make_async_copy(idx_hbm.at[pl.ds(0, C)], ibuf[b], isem[b]).wait()

        def wait_seg(b):
            pltpu.make_async_copy(seg_hbm.at[pl.ds(0, C)], sbuf[b], ssem[b]).wait()

        def wait_data(b):
            pltpu.make_async_copy(af_hbm.at[pl.ds(0, C)], rows[b], gsem[b]).wait()
            pltpu.make_async_copy(filt_hbm.at[pl.ds(0, C)], filt[b], fsem[b]).wait()

        def multiply(b):
            @pl.loop(0, C)
            def _m(r):
                for c8 in range(D // L):
                    sl = pl.ds(c8 * L, L)
                    rows[b][r, sl] = rows[b][r, sl] * filt[b][r, sl]

        for b in range(2):
            issue_idx(b, b)
            issue_seg(b, b)
        for b in range(2):
            wait_idx(b)
            issue_data(b, b)

        NF2 = NF - (NF % 2)

        @pl.loop(0, NF2, step=2)
        def _main(g):
            for b in range(2):
                gg = g + b
                nxt = jnp.minimum(gg + 2, NF - 1)
                wait_data(b)
                issue_idx(nxt, b)
                multiply(b)
                wait_seg(b)
                pltpu.sync_copy(rows[b], acc.at[sbuf[b]], add=True)
                issue_seg(nxt, b)
                wait_idx(b)
                issue_data(nxt, b)

        if NF % 2:
            wait_data(0)
            multiply(0)
            wait_seg(0)
            pltpu.sync_copy(rows[0], acc.at[sbuf[0]], add=True)
            wait_data(1)
            wait_seg(1)
        else:
            for b in range(2):
                wait_data(b)
                wait_seg(b)

        plsc.subcore_barrier()

        pltpu.sync_copy(
            acc.at[pl.ds(sid * RZ, RZ)], out_hbm.at[cid, pl.ds(sid * RZ, RZ)]
        )
        if NREST > 0:
            @pl.when(sid == 0)
            def _rb():
                pltpu.sync_copy(
                    acc.at[pl.ds(RZ * NS, NREST)],
                    out_hbm.at[cid, pl.ds(RZ * NS, NREST)],
                )

    return k(af, filters, idx, seg)


def _add_body(p_ref, q_ref, o_ref):
    o_ref[...] = (p_ref[0] + p_ref[1]) + (q_ref[0] + q_ref[1])


def _add_partials(p, q):
    _, N, D = p.shape
    BN = 2000
    assert N % BN == 0
    spec = pl.BlockSpec((2, BN, D), lambda i: (0, i, 0))
    return pl.pallas_call(
        _add_body,
        grid=(N // BN,),
        in_specs=[spec, spec],
        out_specs=pl.BlockSpec((BN, D), lambda i: (i, 0)),
        out_shape=jax.ShapeDtypeStruct((N, D), jnp.float32),
    )(p, q)


def kernel(atom_features, distances, rbf_centers, rbf_gamma, W1, b1, W2, b2, idx_j, seg_i):
    B, N, D = atom_features.shape
    E = distances.shape[1]
    af = atom_features.reshape(N, D)
    idx = idx_j.astype(jnp.int32)
    seg = seg_i.astype(jnp.int32)

    H = E // 2
    parts = []
    for k in range(2):
        sl = slice(k * H, (k + 1) * H)
        f_k = _compute_filters(
            distances[:, sl], rbf_centers, rbf_gamma, W1, b1, W2, b2)
        parts.append(_sc_gather_mult_segsum(af, f_k, idx[sl], seg[sl]))
    out = _add_partials(parts[0], parts[1])
    return out.reshape(B, N, D)


# speedup vs baseline: 1.0903x; 1.0042x over previous
import functools

import jax
import jax.numpy as jnp
from jax import lax
from jax.experimental import pallas as pl
from jax.experimental.pallas import tpu as pltpu
from jax.experimental.pallas import tpu_sc as plsc

_LN2 = 0.6931471805599453


def _ssp(x):
    return jnp.maximum(x, 0.0) + jnp.log1p(jnp.exp(-jnp.abs(x))) - _LN2


def _filters_body(d_ref, c_ref, g_ref, w1_ref, b1_ref, w2_ref, b2_ref, o_ref):
    d = d_ref[...].reshape(-1, 1)
    ex = jnp.exp(-g_ref[...] * (d - c_ref[...]) ** 2)
    h = jnp.dot(ex, w1_ref[...], preferred_element_type=jnp.float32) + b1_ref[...]
    h = _ssp(h)
    f = jnp.dot(h, w2_ref[...], preferred_element_type=jnp.float32) + b2_ref[...]
    o_ref[...] = _ssp(f)


def _compute_filters(dist, centers, gamma, W1, b1, W2, b2):
    E = dist.shape[1]
    R = centers.shape[0]
    D = W1.shape[1]
    BE = next(b for b in range(3328, 127, -128) if E % b == 0)
    return pl.pallas_call(
        _filters_body,
        grid=(E // BE,),
        in_specs=[
            pl.BlockSpec((1, BE), lambda i: (0, i)),
            pl.BlockSpec((1, R), lambda i: (0, 0)),
            pl.BlockSpec((1, R), lambda i: (0, 0)),
            pl.BlockSpec((R, D), lambda i: (0, 0)),
            pl.BlockSpec((1, D), lambda i: (0, 0)),
            pl.BlockSpec((D, D), lambda i: (0, 0)),
            pl.BlockSpec((1, D), lambda i: (0, 0)),
        ],
        out_specs=pl.BlockSpec((BE, D), lambda i: (i, 0)),
        out_shape=jax.ShapeDtypeStruct((E, D), jnp.float32),
    )(
        dist,
        centers.reshape(1, R),
        gamma.reshape(1, R),
        W1,
        b1.reshape(1, D),
        W2,
        b2.reshape(1, D),
    )


def _sc_gather_mult_segsum(af, filters, idx, seg):
    N, D = af.shape
    E = filters.shape[0]
    NC, NS, L = 2, 16, 16
    NW = NC * NS
    EW = E // NW
    C = next(c for c in range(96, 7, -8) if EW % c == 0)
    assert E % (NW * C) == 0 and D % L == 0
    NF = EW // C
    RZ = (N // NS) // 8 * 8
    NREST = N - RZ * NS
    assert EW % 8 == 0
    mesh = plsc.VectorSubcoreMesh(
        core_axis_name="c", subcore_axis_name="s", num_cores=NC, num_subcores=NS
    )

    @functools.partial(
        pl.kernel,
        out_type=jax.ShapeDtypeStruct((NC, N, D), jnp.float32),
        mesh=mesh,
        scratch_types=[
            pltpu.VMEM((C,), jnp.int32),
            pltpu.VMEM((C,), jnp.int32),
            pltpu.VMEM((C,), jnp.int32),
            pltpu.VMEM((C,), jnp.int32),
            pltpu.VMEM((C, D), jnp.float32),
            pltpu.VMEM((C, D), jnp.float32),
            pltpu.VMEM((C, D), jnp.float32),
            pltpu.VMEM((C, D), jnp.float32),
            pltpu.VMEM_SHARED((N, D), jnp.float32),
            pltpu.SemaphoreType.DMA,
            pltpu.SemaphoreType.DMA,
            pltpu.SemaphoreType.DMA,
            pltpu.SemaphoreType.DMA,
            pltpu.SemaphoreType.DMA,
            pltpu.SemaphoreType.DMA,
            pltpu.SemaphoreType.DMA,
            pltpu.SemaphoreType.DMA,
        ],
    )
    def k(af_hbm, filt_hbm, idx_hbm, seg_hbm, out_hbm,
          ibuf0, ibuf1, sbuf0, sbuf1, rows0, rows1, filt0, filt1, acc,
          gsem0, gsem1, fsem0, fsem1, isem0, isem1, ssem0, ssem1):
        ibuf = (ibuf0, ibuf1)
        sbuf = (sbuf0, sbuf1)
        rows = (rows0, rows1)
        filt = (filt0, filt1)
        gsem = (gsem0, gsem1)
        fsem = (fsem0, fsem1)
        isem = (isem0, isem1)
        ssem = (ssem0, ssem1)
        cid = lax.axis_index("c")
        sid = lax.axis_index("s")
        wid = cid * NS + sid
        ebase = wid * EW

        @pl.loop(0, C)
        def _z(r):
            for c8 in range(D // L):
                rows0[r, pl.ds(c8 * L, L)] = jnp.zeros((L,), jnp.float32)

        nz = RZ // C
        rz = RZ - nz * C

        @pl.loop(0, nz)
        def _zc(kk):
            pltpu.sync_copy(rows0, acc.at[pl.ds(sid * RZ + kk * C, C)])

        if rz > 0:
            pltpu.sync_copy(
                rows0.at[pl.ds(0, rz)], acc.at[pl.ds(sid * RZ + nz * C, rz)]
            )
        if NREST > 0:
            @pl.when(sid == 0)
            def _zrest():
                pltpu.sync_copy(
                    rows0.at[pl.ds(0, NREST)], acc.at[pl.ds(RZ * NS, NREST)]
                )
        plsc.subcore_barrier()

        def issue_idx(g, b):
            pltpu.async_copy(idx_hbm.at[pl.ds(ebase + g * C, C)], ibuf[b], isem[b])

        def issue_seg(g, b):
            pltpu.async_copy(seg_hbm.at[pl.ds(ebase + g * C, C)], sbuf[b], ssem[b])

        def issue_data(g, b):
            pltpu.async_copy(af_hbm.at[ibuf[b]], rows[b], gsem[b])
            pltpu.async_copy(
                filt_hbm.at[pl.ds(ebase + g * C, C)], filt[b], fsem[b]
            )

        def wait_idx(b):
            pltpu.make_async_copy(idx_hbm.at[pl.ds(0, C)], ibuf[b], isem[b]).wait()

        def wait_seg(b):
            pltpu.make_async_copy(seg_hbm.at[pl.ds(0, C)], sbuf[b], ssem[b]).wait()

        def wait_data(b):
            pltpu.make_async_copy(af_hbm.at[pl.ds(0, C)], rows[b], gsem[b]).wait()
            pltpu.make_async_copy(filt_hbm.at[pl.ds(0, C)], filt[b], fsem[b]).wait()

        def multiply(b):
            @pl.loop(0, C)
            def _m(r):
                for c8 in range(D // L):
                    sl = pl.ds(c8 * L, L)
                    rows[b][r, sl] = rows[b][r, sl] * filt[b][r, sl]

        for b in range(2):
            issue_idx(b, b)
            issue_seg(b, b)
        for b in range(2):
            wait_idx(b)
            issue_data(b, b)

        NF2 = NF - (NF % 2)

        @pl.loop(0, NF2, step=2)
        def _main(g):
            for b in range(2):
                gg = g + b
                nxt = jnp.minimum(gg + 2, NF - 1)
                wait_data(b)
                issue_idx(nxt, b)
                multiply(b)
                wait_seg(b)
                pltpu.sync_copy(rows[b], acc.at[sbuf[b]], add=True)
                issue_seg(nxt, b)
                wait_idx(b)
                issue_data(nxt, b)

        if NF % 2:
            wait_data(0)
            multiply(0)
            wait_seg(0)
            pltpu.sync_copy(rows[0], acc.at[sbuf[0]], add=True)
            wait_data(1)
            wait_seg(1)
        else:
            for b in range(2):
                wait_data(b)
                wait_seg(b)

        plsc.subcore_barrier()

        pltpu.sync_copy(
            acc.at[pl.ds(sid * RZ, RZ)], out_hbm.at[cid, pl.ds(sid * RZ, RZ)]
        )
        if NREST > 0:
            @pl.when(sid == 0)
            def _rb():
                pltpu.sync_copy(
                    acc.at[pl.ds(RZ * NS, NREST)],
                    out_hbm.at[cid, pl.ds(RZ * NS, NREST)],
                )

    return k(af, filters, idx, seg)


def _add_body(p_ref, q_ref, o_ref):
    o_ref[...] = (p_ref[0] + p_ref[1]) + (q_ref[0] + q_ref[1])


def _add_partials(p, q):
    _, N, D = p.shape
    BN = 2000
    assert N % BN == 0
    spec = pl.BlockSpec((2, BN, D), lambda i: (0, i, 0))
    return pl.pallas_call(
        _add_body,
        grid=(N // BN,),
        in_specs=[spec, spec],
        out_specs=pl.BlockSpec((BN, D), lambda i: (i, 0)),
        out_shape=jax.ShapeDtypeStruct((N, D), jnp.float32),
    )(p, q)


def kernel(atom_features, distances, rbf_centers, rbf_gamma, W1, b1, W2, b2, idx_j, seg_i):
    B, N, D = atom_features.shape
    E = distances.shape[1]
    af = atom_features.reshape(N, D)
    idx = idx_j.astype(jnp.int32)
    seg = seg_i.astype(jnp.int32)

    H = (E * 121 // 250) // 256 * 256
    parts = []
    for k, sl in enumerate((slice(0, H), slice(H, E))):
        f_k = _compute_filters(
            distances[:, sl], rbf_centers, rbf_gamma, W1, b1, W2, b2)
        parts.append(_sc_gather_mult_segsum(af, f_k, idx[sl], seg[sl]))
    out = _add_partials(parts[0], parts[1])
    return out.reshape(B, N, D)
